# baseline (speedup 1.0000x reference)
import functools

import jax
import jax.numpy as jnp
import numpy as np
from jax.experimental import pallas as pl

_MU = 100.0
_M = 256.0
_NB = 1024
_SHIFT = 32000
_SEP = _NB + _SHIFT
_INV_LOG = 1.0 / float(np.log(_M * _MU + 1.0))

_R = 2000


def _tokenize(x):
    mu = jnp.sign(x) * (jnp.log(jnp.abs(x) * _MU + 1.0) * _INV_LOG)
    v = jnp.clip(mu, -1.0, 1.0)
    v = jnp.floor((v + 1.0) * (_NB / 2))
    return jnp.clip(v, 0.0, _NB - 1).astype(jnp.int32) + _SHIFT


def _body(t_ref, a_ref, o_ref):
    tt = _tokenize(t_ref[...])
    at = _tokenize(a_ref[...])
    sep = jnp.full((_R, 1), _SEP, jnp.int32)
    o_ref[...] = jnp.concatenate([tt, sep, at], axis=1)


@jax.jit
def kernel(tensors, actions):
    n = tensors.shape[0]
    grid = n // _R
    return pl.pallas_call(
        _body,
        grid=(grid,),
        in_specs=[
            pl.BlockSpec((_R, 16), lambda i: (i, 0)),
            pl.BlockSpec((_R, 8), lambda i: (i, 0)),
        ],
        out_specs=pl.BlockSpec((_R, 25), lambda i: (i, 0)),
        out_shape=jax.ShapeDtypeStruct((n, 25), jnp.int32),
    )(tensors, actions)
